# Initial kernel scaffold; baseline (speedup 1.0000x reference)
import jax
import jax.numpy as jnp
from jax.experimental import pallas as pl


def kernel(u, i, c, i1, i2, i3, c1, c2, c3, nv, nf, nc, nb, user_emb, item_emb, cate_emb, hist_emb, W1, b1, W2, b2, W3, b3, W4, b4):
    raise NotImplementedError("write your pallas kernel here")


# speedup vs baseline: 3.8278x; 3.8278x over previous
import jax
import jax.numpy as jnp
from jax.experimental import pallas as pl

B = 16384
EMB = 32
BC = 1024


def _mlp_body(x_ref, w1, b1, w2, b2, w3, b3, w4, b4, out_ref):
  x = x_ref[...]
  h = jnp.maximum(
      jnp.dot(x, w1[...], preferred_element_type=jnp.float32) + b1[...], 0.0)
  h = jnp.maximum(
      jnp.dot(h, w2[...], preferred_element_type=jnp.float32) + b2[...], 0.0)
  h = jnp.maximum(
      jnp.dot(h, w3[...], preferred_element_type=jnp.float32) + b3[...], 0.0)
  z = jnp.dot(h, w4[...], preferred_element_type=jnp.float32) + b4[...]
  out_ref[...] = 1.0 / (1.0 + jnp.exp(-z))


def _mlp(x, W1, b1, W2, b2, W3, b3, W4, b4):
  full = lambda shape: pl.BlockSpec(shape, lambda i: (0,) * len(shape))
  return pl.pallas_call(
      _mlp_body,
      grid=(B // BC,),
      in_specs=[
          pl.BlockSpec((BC, 292), lambda i: (i, 0)),
          full(W1.shape), full((1, 512)),
          full(W2.shape), full((1, 256)),
          full(W3.shape), full((1, 128)),
          full(W4.shape), full((1, 1)),
      ],
      out_specs=pl.BlockSpec((BC, 1), lambda i: (i, 0)),
      out_shape=jax.ShapeDtypeStruct((B, 1), jnp.float32),
  )(x, W1, b1, W2, b2, W3, b3, W4, b4)


def kernel(u, i, c, i1, i2, i3, c1, c2, c3, nv, nf, nc, nb,
           user_emb, item_emb, cate_emb, hist_emb,
           W1, b1, W2, b2, W3, b3, W4, b4):
  x = jnp.concatenate([
      jnp.take(user_emb, u, axis=0),
      jnp.take(item_emb, i, axis=0),
      jnp.take(cate_emb, c, axis=0),
      jnp.take(hist_emb, i1, axis=0),
      jnp.take(hist_emb, i2, axis=0),
      jnp.take(hist_emb, i3, axis=0),
      jnp.take(cate_emb, c1, axis=0),
      jnp.take(cate_emb, c2, axis=0),
      jnp.take(cate_emb, c3, axis=0),
      jnp.stack([nv, nf, nc, nb], axis=1),
  ], axis=1)
  out = _mlp(x, W1, b1.reshape(1, -1), W2, b2.reshape(1, -1),
             W3, b3.reshape(1, -1), W4, b4.reshape(1, -1))
  return out[:, 0]
